# baseline (speedup 1.0000x reference)
import functools

import jax
import jax.numpy as jnp
from jax import lax
from jax.experimental import pallas as pl
from jax.experimental.pallas import tpu as pltpu
from jax.experimental.pallas import tpu_sc as plsc

E = 8
TOP_K = 2
D = 1024
FF = 1408
SFF = 5632
T = 2048
BT = 256
NPAD = T * TOP_K + E * BT
NB = NPAD // BT
NW = 32


def _router_body(x_ref, gw_ref, logits_ref, w_ref, sel_ref):
    x = x_ref[...]
    logits = jnp.dot(x, gw_ref[...], preferred_element_type=jnp.float32)
    logits_ref[...] = logits
    m = jnp.max(logits, axis=-1, keepdims=True)
    ex = jnp.exp(logits - m)
    probs = ex / jnp.sum(ex, axis=-1, keepdims=True)
    i1 = jnp.argmax(probs, axis=-1)
    p1 = jnp.max(probs, axis=-1)
    cols = lax.broadcasted_iota(jnp.int32, probs.shape, 1)
    masked = jnp.where(cols == i1[:, None], -jnp.inf, probs)
    i2 = jnp.argmax(masked, axis=-1)
    p2 = jnp.max(masked, axis=-1)
    denom = p1 + p2
    w_ref[...] = jnp.stack([p1 / denom, p2 / denom], axis=-1)
    sel_ref[...] = jnp.stack([i1, i2], axis=-1).astype(jnp.int32)


def _router(x, gate_w):
    return pl.pallas_call(
        _router_body,
        out_shape=(
            jax.ShapeDtypeStruct((T, E), jnp.float32),
            jax.ShapeDtypeStruct((T, TOP_K), jnp.float32),
            jax.ShapeDtypeStruct((T, TOP_K), jnp.int32),
        ),
        name="tc_router",
    )(x, gate_w)


def _gmm_body(be_ref, na_ref, xs_ref, wg_ref, wu_ref, wd_ref, sw_ref, ys_ref):
    i = pl.program_id(0)

    @pl.when(i < na_ref[0])
    def _():
        xb = xs_ref[...].astype(jnp.bfloat16)
        g = jnp.dot(xb, wg_ref[0].astype(jnp.bfloat16),
                    preferred_element_type=jnp.float32)
        u = jnp.dot(xb, wu_ref[0].astype(jnp.bfloat16),
                    preferred_element_type=jnp.float32)
        h = ((g * jax.nn.sigmoid(g)) * u).astype(jnp.bfloat16)
        y = jnp.dot(h, wd_ref[0].astype(jnp.bfloat16),
                    preferred_element_type=jnp.float32)
        ys_ref[...] = y * sw_ref[...]

    @pl.when(i >= na_ref[0])
    def _():
        ys_ref[...] = jnp.zeros_like(ys_ref)


def _gmm(xs, Wg, Wu, Wd, slot_w, block_expert, n_active):
    grid_spec = pltpu.PrefetchScalarGridSpec(
        num_scalar_prefetch=2,
        grid=(NB,),
        in_specs=[
            pl.BlockSpec((BT, D), lambda i, be, na: (i, 0)),
            pl.BlockSpec((1, D, FF), lambda i, be, na: (be[i], 0, 0)),
            pl.BlockSpec((1, D, FF), lambda i, be, na: (be[i], 0, 0)),
            pl.BlockSpec((1, FF, D), lambda i, be, na: (be[i], 0, 0)),
            pl.BlockSpec((BT, 1), lambda i, be, na: (i, 0)),
        ],
        out_specs=pl.BlockSpec((BT, D), lambda i, be, na: (i, 0)),
    )
    return pl.pallas_call(
        _gmm_body,
        grid_spec=grid_spec,
        out_shape=jax.ShapeDtypeStruct((NPAD, D), jnp.float32),
        name="tc_gmm",
    )(block_expert, n_active, xs, Wg, Wu, Wd, slot_w.reshape(NPAD, 1))


_SFB = 512
_NSF = SFF // _SFB


def _shared_body(x_ref, sg_ref, su_ref, sd_ref, segw_ref, o_ref):
    f = pl.program_id(0)
    xb = x_ref[...]
    g = jnp.dot(xb, sg_ref[...], preferred_element_type=jnp.float32)
    u = jnp.dot(xb, su_ref[...], preferred_element_type=jnp.float32)
    h = (g * jax.nn.sigmoid(g)) * u
    y = jnp.dot(h, sd_ref[...], preferred_element_type=jnp.float32)

    @pl.when(f == 0)
    def _():
        o_ref[...] = y

    @pl.when(f > 0)
    def _():
        o_ref[...] = o_ref[...] + y

    @pl.when(f == _NSF - 1)
    def _():
        seg = jnp.dot(xb, segw_ref[...], preferred_element_type=jnp.float32)
        o_ref[...] = o_ref[...] * jax.nn.sigmoid(seg)


def _shared(x, Sg, Su, Sd, seg_w):
    return pl.pallas_call(
        _shared_body,
        grid=(_NSF,),
        in_specs=[
            pl.BlockSpec((T, D), lambda f: (0, 0)),
            pl.BlockSpec((D, _SFB), lambda f: (0, f)),
            pl.BlockSpec((D, _SFB), lambda f: (0, f)),
            pl.BlockSpec((_SFB, D), lambda f: (f, 0)),
            pl.BlockSpec((D, 1), lambda f: (0, 0)),
        ],
        out_specs=pl.BlockSpec((T, D), lambda f: (0, 0)),
        out_shape=jax.ShapeDtypeStruct((T, D), jnp.float32),
        name="tc_shared",
    )(x, Sg, Su, Sd, seg_w)


def _final_add_body(sh_ref, z0_ref, z1_ref, o_ref):
    o_ref[...] = sh_ref[...] + z0_ref[...] + z1_ref[...]


def _final_add(shared_out, z):
    nt = T // 256
    return pl.pallas_call(
        _final_add_body,
        grid=(nt,),
        in_specs=[
            pl.BlockSpec((256, D), lambda t: (t, 0)),
            pl.BlockSpec((256, D), lambda t: (t, 0)),
            pl.BlockSpec((256, D), lambda t: (t + nt, 0)),
        ],
        out_specs=pl.BlockSpec((256, D), lambda t: (t, 0)),
        out_shape=jax.ShapeDtypeStruct((T, D), jnp.float32),
        name="tc_final_add",
    )(shared_out, z, z)


def _make_sc_gather(n_rows, chunk, name):
    bpw = n_rows // NW
    nc = bpw // chunk

    def body(x_hbm, tok_hbm, out_hbm, idx_v, a_v, b_v,
             ga_sem, gb_sem, oa_sem, ob_sem):
        wid = lax.axis_index("s") * 2 + lax.axis_index("c")
        base = wid * bpw
        pltpu.sync_copy(tok_hbm.at[pl.ds(base, bpw)], idx_v)
        bufs = (a_v, b_v)
        gsems = (ga_sem, gb_sem)
        osems = (oa_sem, ob_sem)

        def start_gather(c):
            return pltpu.async_copy(
                x_hbm.at[idx_v.at[pl.ds(c * chunk, chunk)]], bufs[c % 2],
                gsems[c % 2])

        def start_out(c):
            return pltpu.async_copy(
                bufs[c % 2], out_hbm.at[pl.ds(base + c * chunk, chunk)],
                osems[c % 2])

        g = [start_gather(0), start_gather(1) if nc > 1 else None]
        o = [None, None]
        for c in range(nc):
            b = c % 2
            g[b].wait()
            o[b] = start_out(c)
            if c + 2 < nc:
                o[b].wait()
                g[b] = start_gather(c + 2)
        for h in o:
            if h is not None:
                h.wait()

    def gather(table, idx):
        mesh = plsc.VectorSubcoreMesh(core_axis_name="c", subcore_axis_name="s")
        return pl.kernel(
            body,
            out_type=jax.ShapeDtypeStruct((n_rows, D), jnp.float32),
            mesh=mesh,
            scratch_types=[
                pltpu.VMEM((bpw,), jnp.int32),
                pltpu.VMEM((chunk, D), jnp.float32),
                pltpu.VMEM((chunk, D), jnp.float32),
                pltpu.SemaphoreType.DMA,
                pltpu.SemaphoreType.DMA,
                pltpu.SemaphoreType.DMA,
                pltpu.SemaphoreType.DMA,
            ],
            name=name,
        )(table, idx)

    return gather


_sc_gather = _make_sc_gather(NPAD, 32, "sc_dispatch")
_sc_gather_back = _make_sc_gather(2 * T, 32, "sc_return")


def kernel(hidden_states, gate_w, Wg, Wu, Wd, Sg, Su, Sd, seg_w):
    b, s, dm = hidden_states.shape
    x = hidden_states.reshape(-1, dm)

    router_logits, w, sel = _router(x, gate_w)

    mi = jnp.sum(
        (sel[:, :, None] == jnp.arange(E, dtype=jnp.int32)[None, None, :])
        .astype(jnp.int32), axis=1)
    counts = jnp.sum(mi, axis=0)
    rank = jnp.cumsum(mi, axis=0) - mi
    padded = ((counts + BT - 1) // BT) * BT
    ends = jnp.cumsum(padded)
    offs = ends - padded
    pos = (jnp.take(offs, sel, axis=0)
           + jnp.take_along_axis(rank, sel, axis=1)).astype(jnp.int32)
    block_expert = jnp.minimum(
        jnp.searchsorted(ends, jnp.arange(NB, dtype=jnp.int32) * BT,
                         side="right"),
        E - 1).astype(jnp.int32)
    n_active = ((ends[-1] + BT - 1) // BT).astype(jnp.int32).reshape(1)

    na = T * TOP_K
    slot_asgn = jnp.full((NPAD,), na, jnp.int32).at[pos.reshape(-1)].set(
        jnp.arange(na, dtype=jnp.int32))
    slot_tok = jnp.where(slot_asgn == na,
                         jnp.arange(NPAD, dtype=jnp.int32) % T,
                         slot_asgn >> 1)
    slot_w = jnp.concatenate([w.reshape(-1),
                              jnp.zeros((1,), jnp.float32)])[slot_asgn]

    xs = _sc_gather(x, slot_tok)
    ys = _gmm(xs, Wg, Wu, Wd, slot_w, block_expert, n_active)
    z = _sc_gather_back(ys, pos.T.reshape(-1))
    shared_out = _shared(x, Sg, Su, Sd, seg_w)
    out = _final_add(shared_out, z)

    return out.reshape(b, s, dm), router_logits


# speedup vs baseline: 1.0393x; 1.0393x over previous
import functools

import jax
import jax.numpy as jnp
from jax import lax
from jax.experimental import pallas as pl
from jax.experimental.pallas import tpu as pltpu
from jax.experimental.pallas import tpu_sc as plsc

E = 8
TOP_K = 2
D = 1024
FF = 1408
SFF = 5632
T = 2048
BT = 256
NPAD = T * TOP_K + E * BT
NB = NPAD // BT
NW = 32


def _router_body(x_ref, gw_ref, logits_ref, w_ref, sel_ref):
    x = x_ref[...]
    logits = jnp.dot(x, gw_ref[...], preferred_element_type=jnp.float32)
    logits_ref[...] = logits
    m = jnp.max(logits, axis=-1, keepdims=True)
    ex = jnp.exp(logits - m)
    probs = ex / jnp.sum(ex, axis=-1, keepdims=True)
    i1 = jnp.argmax(probs, axis=-1)
    p1 = jnp.max(probs, axis=-1)
    cols = lax.broadcasted_iota(jnp.int32, probs.shape, 1)
    masked = jnp.where(cols == i1[:, None], -jnp.inf, probs)
    i2 = jnp.argmax(masked, axis=-1)
    p2 = jnp.max(masked, axis=-1)
    denom = p1 + p2
    w_ref[...] = jnp.stack([p1 / denom, p2 / denom], axis=-1)
    sel_ref[...] = jnp.stack([i1, i2], axis=-1).astype(jnp.int32)


def _router(x, gate_w):
    return pl.pallas_call(
        _router_body,
        out_shape=(
            jax.ShapeDtypeStruct((T, E), jnp.float32),
            jax.ShapeDtypeStruct((T, TOP_K), jnp.float32),
            jax.ShapeDtypeStruct((T, TOP_K), jnp.int32),
        ),
        name="tc_router",
    )(x, gate_w)


def _gmm_body(be_ref, na_ref, xs_ref, wg_ref, wu_ref, wd_ref, sw_ref, ys_ref):
    i = pl.program_id(0)

    @pl.when(i < na_ref[0])
    def _():
        xb = xs_ref[...].astype(jnp.bfloat16)
        g = jnp.dot(xb, wg_ref[0].astype(jnp.bfloat16),
                    preferred_element_type=jnp.float32)
        u = jnp.dot(xb, wu_ref[0].astype(jnp.bfloat16),
                    preferred_element_type=jnp.float32)
        h = ((g * jax.nn.sigmoid(g)) * u).astype(jnp.bfloat16)
        y = jnp.dot(h, wd_ref[0].astype(jnp.bfloat16),
                    preferred_element_type=jnp.float32)
        ys_ref[...] = y * sw_ref[...]

    @pl.when(i >= na_ref[0])
    def _():
        ys_ref[...] = jnp.zeros_like(ys_ref)


def _gmm(xs, Wg, Wu, Wd, slot_w, block_expert, n_active):
    grid_spec = pltpu.PrefetchScalarGridSpec(
        num_scalar_prefetch=2,
        grid=(NB,),
        in_specs=[
            pl.BlockSpec((BT, D), lambda i, be, na: (i, 0)),
            pl.BlockSpec((1, D, FF), lambda i, be, na: (be[i], 0, 0)),
            pl.BlockSpec((1, D, FF), lambda i, be, na: (be[i], 0, 0)),
            pl.BlockSpec((1, FF, D), lambda i, be, na: (be[i], 0, 0)),
            pl.BlockSpec((BT, 1), lambda i, be, na: (i, 0)),
        ],
        out_specs=pl.BlockSpec((BT, D), lambda i, be, na: (i, 0)),
    )
    return pl.pallas_call(
        _gmm_body,
        grid_spec=grid_spec,
        out_shape=jax.ShapeDtypeStruct((NPAD, D), jnp.float32),
        name="tc_gmm",
    )(block_expert, n_active, xs, Wg, Wu, Wd, slot_w.reshape(NPAD, 1))


_SFB = 512
_NSF = SFF // _SFB


def _shared_body(x_ref, sg_ref, su_ref, sd_ref, segw_ref, o_ref):
    f = pl.program_id(0)
    xb = x_ref[...]
    g = jnp.dot(xb, sg_ref[...], preferred_element_type=jnp.float32)
    u = jnp.dot(xb, su_ref[...], preferred_element_type=jnp.float32)
    h = (g * jax.nn.sigmoid(g)) * u
    y = jnp.dot(h, sd_ref[...], preferred_element_type=jnp.float32)

    @pl.when(f == 0)
    def _():
        o_ref[...] = y

    @pl.when(f > 0)
    def _():
        o_ref[...] = o_ref[...] + y

    @pl.when(f == _NSF - 1)
    def _():
        seg = jnp.dot(xb, segw_ref[...], preferred_element_type=jnp.float32)
        o_ref[...] = o_ref[...] * jax.nn.sigmoid(seg)


def _shared(x, Sg, Su, Sd, seg_w):
    return pl.pallas_call(
        _shared_body,
        grid=(_NSF,),
        in_specs=[
            pl.BlockSpec((T, D), lambda f: (0, 0)),
            pl.BlockSpec((D, _SFB), lambda f: (0, f)),
            pl.BlockSpec((D, _SFB), lambda f: (0, f)),
            pl.BlockSpec((_SFB, D), lambda f: (f, 0)),
            pl.BlockSpec((D, 1), lambda f: (0, 0)),
        ],
        out_specs=pl.BlockSpec((T, D), lambda f: (0, 0)),
        out_shape=jax.ShapeDtypeStruct((T, D), jnp.float32),
        name="tc_shared",
    )(x, Sg, Su, Sd, seg_w)


def _final_add_body(sh_ref, z0_ref, z1_ref, o_ref):
    o_ref[...] = sh_ref[...] + z0_ref[...] + z1_ref[...]


def _final_add(shared_out, z):
    nt = T // 256
    return pl.pallas_call(
        _final_add_body,
        grid=(nt,),
        in_specs=[
            pl.BlockSpec((256, D), lambda t: (t, 0)),
            pl.BlockSpec((256, D), lambda t: (t, 0)),
            pl.BlockSpec((256, D), lambda t: (t + nt, 0)),
        ],
        out_specs=pl.BlockSpec((256, D), lambda t: (t, 0)),
        out_shape=jax.ShapeDtypeStruct((T, D), jnp.float32),
        name="tc_final_add",
    )(shared_out, z, z)


def _make_sc_gather(n_rows, chunk, name):
    bpw = n_rows // NW
    nc = bpw // chunk

    def body(x_hbm, tok_hbm, out_hbm, idx_v, a_v, b_v,
             ga_sem, gb_sem, oa_sem, ob_sem):
        wid = lax.axis_index("s") * 2 + lax.axis_index("c")
        base = wid * bpw
        pltpu.sync_copy(tok_hbm.at[pl.ds(base, bpw)], idx_v)
        bufs = (a_v, b_v)
        gsems = (ga_sem, gb_sem)
        osems = (oa_sem, ob_sem)

        def start_gather(c):
            return pltpu.async_copy(
                x_hbm.at[idx_v.at[pl.ds(c * chunk, chunk)]], bufs[c % 2],
                gsems[c % 2])

        def start_out(c):
            return pltpu.async_copy(
                bufs[c % 2], out_hbm.at[pl.ds(base + c * chunk, chunk)],
                osems[c % 2])

        g = [start_gather(0), start_gather(1) if nc > 1 else None]
        o = [None, None]
        for c in range(nc):
            b = c % 2
            g[b].wait()
            o[b] = start_out(c)
            if c + 2 < nc:
                o[b].wait()
                g[b] = start_gather(c + 2)
        for h in o:
            if h is not None:
                h.wait()

    def gather(table, idx):
        mesh = plsc.VectorSubcoreMesh(core_axis_name="c", subcore_axis_name="s")
        return pl.kernel(
            body,
            out_type=jax.ShapeDtypeStruct((n_rows, D), jnp.float32),
            mesh=mesh,
            scratch_types=[
                pltpu.VMEM((bpw,), jnp.int32),
                pltpu.VMEM((chunk, D), jnp.float32),
                pltpu.VMEM((chunk, D), jnp.float32),
                pltpu.SemaphoreType.DMA,
                pltpu.SemaphoreType.DMA,
                pltpu.SemaphoreType.DMA,
                pltpu.SemaphoreType.DMA,
            ],
            name=name,
        )(table, idx)

    return gather


_sc_gather = _make_sc_gather(NPAD, 32, "sc_dispatch")
_sc_gather_back = _make_sc_gather(2 * T, 32, "sc_return")


def kernel(hidden_states, gate_w, Wg, Wu, Wd, Sg, Su, Sd, seg_w):
    b, s, dm = hidden_states.shape
    x = hidden_states.reshape(-1, dm)

    router_logits, w, sel = _router(x, gate_w)

    mi = jnp.sum(
        (sel[:, :, None] == jnp.arange(E, dtype=jnp.int32)[None, None, :])
        .astype(jnp.int32), axis=1)
    counts = jnp.sum(mi, axis=0)
    rank = jnp.cumsum(mi, axis=0) - mi
    padded = ((counts + BT - 1) // BT) * BT
    ends = jnp.cumsum(padded)
    offs = ends - padded
    pos = (jnp.take(offs, sel, axis=0)
           + jnp.take_along_axis(rank, sel, axis=1)).astype(jnp.int32)
    block_expert = jnp.minimum(
        jnp.searchsorted(ends, jnp.arange(NB, dtype=jnp.int32) * BT,
                         side="right"),
        E - 1).astype(jnp.int32)
    n_active = ((ends[-1] + BT - 1) // BT).astype(jnp.int32).reshape(1)

    flat_pos = pos.reshape(-1)
    slot_tok = (jnp.arange(NPAD, dtype=jnp.int32) % T).at[flat_pos].set(
        jnp.repeat(jnp.arange(T, dtype=jnp.int32), TOP_K))
    slot_w = jnp.zeros((NPAD,), jnp.float32).at[flat_pos].set(w.reshape(-1))

    xs = _sc_gather(x, slot_tok)
    ys = _gmm(xs, Wg, Wu, Wd, slot_w, block_expert, n_active)
    z = _sc_gather_back(ys, pos.T.reshape(-1))
    shared_out = _shared(x, Sg, Su, Sd, seg_w)
    out = _final_add(shared_out, z)

    return out.reshape(b, s, dm), router_logits


# speedup vs baseline: 1.0449x; 1.0054x over previous
import jax
import jax.numpy as jnp
from jax import lax
from jax.experimental import pallas as pl
from jax.experimental.pallas import tpu as pltpu
from jax.experimental.pallas import tpu_sc as plsc

E = 8
TOP_K = 2
D = 1024
FF = 1408
SFF = 5632
T = 2048
BT = 256
NPAD = T * TOP_K + E * BT
NB = NPAD // BT
NW = 32


def _router_body(x_ref, gw_ref, logits_ref, w_ref, sel_ref):
    x = x_ref[...]
    logits = jnp.dot(x, gw_ref[...], preferred_element_type=jnp.float32)
    logits_ref[...] = logits
    m = jnp.max(logits, axis=-1, keepdims=True)
    ex = jnp.exp(logits - m)
    probs = ex / jnp.sum(ex, axis=-1, keepdims=True)
    i1 = jnp.argmax(probs, axis=-1)
    p1 = jnp.max(probs, axis=-1)
    cols = lax.broadcasted_iota(jnp.int32, probs.shape, 1)
    masked = jnp.where(cols == i1[:, None], -jnp.inf, probs)
    i2 = jnp.argmax(masked, axis=-1)
    p2 = jnp.max(masked, axis=-1)
    denom = p1 + p2
    w_ref[...] = jnp.stack([p1 / denom, p2 / denom], axis=-1)
    sel_ref[...] = jnp.stack([i1, i2], axis=-1).astype(jnp.int32)


def _router(x, gate_w):
    return pl.pallas_call(
        _router_body,
        out_shape=(
            jax.ShapeDtypeStruct((T, E), jnp.float32),
            jax.ShapeDtypeStruct((T, TOP_K), jnp.float32),
            jax.ShapeDtypeStruct((T, TOP_K), jnp.int32),
        ),
        name="tc_router",
    )(x, gate_w)


def _gmm_body(be_ref, na_ref, xs_ref, wg_ref, wu_ref, wd_ref, sw_ref, ys_ref):
    i = pl.program_id(0)

    @pl.when(i < na_ref[0])
    def _():
        xb = xs_ref[...].astype(jnp.bfloat16)
        g = jnp.dot(xb, wg_ref[0].astype(jnp.bfloat16),
                    preferred_element_type=jnp.float32)
        u = jnp.dot(xb, wu_ref[0].astype(jnp.bfloat16),
                    preferred_element_type=jnp.float32)
        h = ((g * jax.nn.sigmoid(g)) * u).astype(jnp.bfloat16)
        y = jnp.dot(h, wd_ref[0].astype(jnp.bfloat16),
                    preferred_element_type=jnp.float32)
        ys_ref[...] = y * sw_ref[...]


def _gmm(xs, Wg, Wu, Wd, slot_w, block_expert, n_active):
    grid_spec = pltpu.PrefetchScalarGridSpec(
        num_scalar_prefetch=2,
        grid=(NB,),
        in_specs=[
            pl.BlockSpec((BT, D), lambda i, be, na: (i, 0)),
            pl.BlockSpec((1, D, FF), lambda i, be, na: (be[i], 0, 0)),
            pl.BlockSpec((1, D, FF), lambda i, be, na: (be[i], 0, 0)),
            pl.BlockSpec((1, FF, D), lambda i, be, na: (be[i], 0, 0)),
            pl.BlockSpec((BT, 1), lambda i, be, na: (i, 0)),
        ],
        out_specs=pl.BlockSpec((BT, D), lambda i, be, na: (i, 0)),
    )
    return pl.pallas_call(
        _gmm_body,
        grid_spec=grid_spec,
        out_shape=jax.ShapeDtypeStruct((NPAD, D), jnp.float32),
        name="tc_gmm",
    )(block_expert, n_active, xs, Wg, Wu, Wd, slot_w.reshape(NPAD, 1))


_SFB = 512
_NSF = SFF // _SFB


def _shared_body(x_ref, sg_ref, su_ref, sd_ref, segw_ref, o_ref):
    f = pl.program_id(0)
    xb = x_ref[...]
    g = jnp.dot(xb, sg_ref[...], preferred_element_type=jnp.float32)
    u = jnp.dot(xb, su_ref[...], preferred_element_type=jnp.float32)
    h = (g * jax.nn.sigmoid(g)) * u
    y = jnp.dot(h, sd_ref[...], preferred_element_type=jnp.float32)

    @pl.when(f == 0)
    def _():
        o_ref[...] = y

    @pl.when(f > 0)
    def _():
        o_ref[...] = o_ref[...] + y

    @pl.when(f == _NSF - 1)
    def _():
        seg = jnp.dot(xb, segw_ref[...], preferred_element_type=jnp.float32)
        o_ref[...] = o_ref[...] * jax.nn.sigmoid(seg)


def _shared(x, Sg, Su, Sd, seg_w):
    return pl.pallas_call(
        _shared_body,
        grid=(_NSF,),
        in_specs=[
            pl.BlockSpec((T, D), lambda f: (0, 0)),
            pl.BlockSpec((D, _SFB), lambda f: (0, f)),
            pl.BlockSpec((D, _SFB), lambda f: (0, f)),
            pl.BlockSpec((_SFB, D), lambda f: (f, 0)),
            pl.BlockSpec((D, 1), lambda f: (0, 0)),
        ],
        out_specs=pl.BlockSpec((T, D), lambda f: (0, 0)),
        out_shape=jax.ShapeDtypeStruct((T, D), jnp.float32),
        name="tc_shared",
    )(x, Sg, Su, Sd, seg_w)


def _final_add_body(sh_ref, z0_ref, z1_ref, o_ref):
    o_ref[...] = sh_ref[...] + z0_ref[...] + z1_ref[...]


def _final_add(shared_out, z):
    nt = T // 256
    return pl.pallas_call(
        _final_add_body,
        grid=(nt,),
        in_specs=[
            pl.BlockSpec((256, D), lambda t: (t, 0)),
            pl.BlockSpec((256, D), lambda t: (t, 0)),
            pl.BlockSpec((256, D), lambda t: (t + nt, 0)),
        ],
        out_specs=pl.BlockSpec((256, D), lambda t: (t, 0)),
        out_shape=jax.ShapeDtypeStruct((T, D), jnp.float32),
        name="tc_final_add",
    )(shared_out, z, z)


def _make_sc_gather(n_rows, chunk, name):
    bpw = n_rows // NW
    nc = bpw // chunk

    def body(x_hbm, tok_hbm, out_hbm, idx_v, a_v, b_v,
             ga_sem, gb_sem, oa_sem, ob_sem):
        wid = lax.axis_index("s") * 2 + lax.axis_index("c")
        base = wid * bpw
        pltpu.sync_copy(tok_hbm.at[pl.ds(base, bpw)], idx_v)
        bufs = (a_v, b_v)
        gsems = (ga_sem, gb_sem)
        osems = (oa_sem, ob_sem)

        def start_gather(c):
            return pltpu.async_copy(
                x_hbm.at[idx_v.at[pl.ds(c * chunk, chunk)]], bufs[c % 2],
                gsems[c % 2])

        def start_out(c):
            return pltpu.async_copy(
                bufs[c % 2], out_hbm.at[pl.ds(base + c * chunk, chunk)],
                osems[c % 2])

        g = [start_gather(0), start_gather(1) if nc > 1 else None]
        o = [None, None]
        for c in range(nc):
            b = c % 2
            g[b].wait()
            o[b] = start_out(c)
            if c + 2 < nc:
                o[b].wait()
                g[b] = start_gather(c + 2)
        for h in o:
            if h is not None:
                h.wait()

    def gather(table, idx):
        mesh = plsc.VectorSubcoreMesh(core_axis_name="c", subcore_axis_name="s")
        return pl.kernel(
            body,
            out_type=jax.ShapeDtypeStruct((n_rows, D), jnp.float32),
            mesh=mesh,
            scratch_types=[
                pltpu.VMEM((bpw,), jnp.int32),
                pltpu.VMEM((chunk, D), jnp.float32),
                pltpu.VMEM((chunk, D), jnp.float32),
                pltpu.SemaphoreType.DMA,
                pltpu.SemaphoreType.DMA,
                pltpu.SemaphoreType.DMA,
                pltpu.SemaphoreType.DMA,
            ],
            name=name,
        )(table, idx)

    return gather


_sc_gather = _make_sc_gather(NPAD, 32, "sc_dispatch")
_sc_gather_back = _make_sc_gather(2 * T, 32, "sc_return")


def kernel(hidden_states, gate_w, Wg, Wu, Wd, Sg, Su, Sd, seg_w):
    b, s, dm = hidden_states.shape
    x = hidden_states.reshape(-1, dm)

    router_logits, w, sel = _router(x, gate_w)

    mi = jnp.sum(
        (sel[:, :, None] == jnp.arange(E, dtype=jnp.int32)[None, None, :])
        .astype(jnp.int32), axis=1)
    counts = jnp.sum(mi, axis=0)
    rank = jnp.cumsum(mi, axis=0) - mi
    padded = ((counts + BT - 1) // BT) * BT
    ends = jnp.cumsum(padded)
    offs = ends - padded
    pos = (jnp.take(offs, sel, axis=0)
           + jnp.take_along_axis(rank, sel, axis=1)).astype(jnp.int32)
    block_expert = jnp.minimum(
        jnp.searchsorted(ends, jnp.arange(NB, dtype=jnp.int32) * BT,
                         side="right"),
        E - 1).astype(jnp.int32)
    n_active = ((ends[-1] + BT - 1) // BT).astype(jnp.int32).reshape(1)

    flat_pos = pos.reshape(-1)
    slot_tok = (jnp.arange(NPAD, dtype=jnp.int32) % T).at[flat_pos].set(
        jnp.repeat(jnp.arange(T, dtype=jnp.int32), TOP_K))
    slot_w = jnp.zeros((NPAD,), jnp.float32).at[flat_pos].set(w.reshape(-1))

    xs = _sc_gather(x, slot_tok)
    ys = _gmm(xs, Wg, Wu, Wd, slot_w, block_expert, n_active)
    z = _sc_gather_back(ys, pos.T.reshape(-1))
    shared_out = _shared(x, Sg, Su, Sd, seg_w)
    out = _final_add(shared_out, z)

    return out.reshape(b, s, dm), router_logits
